# baseline (speedup 1.0000x reference)
import functools

import jax
import jax.numpy as jnp
from jax import lax
from jax.experimental import pallas as pl
from jax.experimental.pallas import tpu as pltpu
from jax.experimental.pallas import tpu_sc as plsc

N_EDGES = 800000
EMB = 64
NC = 2
NS = 16
NW = NC * NS
CHUNK = 640
NCHUNKS = N_EDGES // CHUNK
KMAX = (NCHUNKS + NW - 1) // NW
NGROUP = CHUNK // 16
CROWS = 126


def _body(c_hbm, a0_hbm, a1_hbm, a2_hbm, out_hbm,
          c_v, av0, av1, av2, rows_v,
          s_idx0, s_idx1, s_o0, s_o1):
    wid = lax.axis_index("s") * NC + lax.axis_index("c")
    s_idx = (s_idx0, s_idx1)
    s_o = (s_o0, s_o1)
    iota = lax.iota(jnp.int32, 16)

    pltpu.sync_copy(c_hbm, c_v)

    def stage_idx(c, p):
        off = pl.multiple_of(c * CHUNK, 8)
        pltpu.async_copy(a0_hbm.at[pl.ds(off, CHUNK)], av0.at[p], s_idx[p])
        pltpu.async_copy(a1_hbm.at[pl.ds(off, CHUNK)], av1.at[p], s_idx[p])
        pltpu.async_copy(a2_hbm.at[pl.ds(off, CHUNK)], av2.at[p], s_idx[p])

    def wait_idx(c, p):
        off = pl.multiple_of(c * CHUNK, 8)
        pltpu.make_async_copy(a0_hbm.at[pl.ds(off, CHUNK)], av0.at[p],
                              s_idx[p]).wait()
        pltpu.make_async_copy(a1_hbm.at[pl.ds(off, CHUNK)], av1.at[p],
                              s_idx[p]).wait()
        pltpu.make_async_copy(a2_hbm.at[pl.ds(off, CHUNK)], av2.at[p],
                              s_idx[p]).wait()

    def out_slice(c):
        return out_hbm.at[pl.ds(pl.multiple_of(c * CHUNK * EMB, 8),
                                CHUNK * EMB)]

    def half(i, h):
        k = i * 2 + h
        c = wid + NW * k
        p = h

        @pl.when(c < NCHUNKS)
        def _():
            wait_idx(c, p)
            @pl.when(c + NW < NCHUNKS)
            def _():
                stage_idx(c + NW, 1 - p)
            @pl.when(k >= 2)
            def _():
                pltpu.make_async_copy(rows_v.at[p], out_slice(c - 2 * NW),
                                      s_o[p]).wait()

            rows_p = rows_v.at[p]

            @plsc.parallel_loop(0, NGROUP, step=1, unroll=2)
            def group(g):
                g16 = pl.multiple_of(g * 16, 16)
                sl = pl.ds(g16, 16)
                c0 = jnp.clip(av0.at[p][sl], 0, 5)
                c1 = jnp.clip(av1.at[p][sl], 0, 6)
                c2 = jnp.clip(av2.at[p][sl], 0, 2)
                key = c0 + c1 * 6 + c2 * 42
                gkey = key * EMB
                sbase = (g16 + iota) * EMB
                for b in range(0, EMB, 16):
                    vs = [plsc.load_gather(c_v, [gkey + (b + j)])
                          for j in range(16)]
                    for j in range(16):
                        plsc.store_scatter(rows_p, [sbase + (b + j)], vs[j])

            pltpu.async_copy(rows_p, out_slice(c), s_o[p])

    stage_idx(wid, 0)

    def pair(i, carry):
        half(i, 0)
        half(i, 1)
        return carry

    lax.fori_loop(0, KMAX // 2, pair, 0)

    for k in (KMAX - 2, KMAX - 1):
        c = wid + NW * k

        @pl.when(c < NCHUNKS)
        def _():
            pltpu.make_async_copy(rows_v.at[k % 2], out_slice(c),
                                  s_o[k % 2]).wait()


@functools.partial(jax.jit, donate_argnums=())
def kernel(edge_attr, W0, W1, W2):
    comb = (W2[:, None, None, :] + W1[None, :, None, :]
            + W0[None, None, :, :]).reshape(CROWS * EMB)
    ea = edge_attr.astype(jnp.int32)
    a0, a1, a2 = ea[:, 0], ea[:, 1], ea[:, 2]

    run = pl.kernel(
        _body,
        out_type=jax.ShapeDtypeStruct((N_EDGES * EMB,), jnp.float32),
        mesh=plsc.VectorSubcoreMesh(core_axis_name="c", subcore_axis_name="s"),
        scratch_types=[
            pltpu.VMEM((CROWS * EMB,), jnp.float32),
            pltpu.VMEM((2, CHUNK), jnp.int32),
            pltpu.VMEM((2, CHUNK), jnp.int32),
            pltpu.VMEM((2, CHUNK), jnp.int32),
            pltpu.VMEM((2, CHUNK * EMB), jnp.float32),
            pltpu.SemaphoreType.DMA,
            pltpu.SemaphoreType.DMA,
            pltpu.SemaphoreType.DMA,
            pltpu.SemaphoreType.DMA,
        ],
        compiler_params=pltpu.CompilerParams(
            needs_layout_passes=False, use_tc_tiling_on_sc=False),
    )
    return run(comb, a0, a1, a2).reshape(N_EDGES, EMB)


# speedup vs baseline: 1.0018x; 1.0018x over previous
import functools

import jax
import jax.numpy as jnp
from jax import lax
from jax.experimental import pallas as pl
from jax.experimental.pallas import tpu as pltpu
from jax.experimental.pallas import tpu_sc as plsc

N_EDGES = 800000
EMB = 64
NC = 2
NS = 16
NW = NC * NS
CHUNK = 640
NCHUNKS = N_EDGES // CHUNK
KMAX = (NCHUNKS + NW - 1) // NW
NGROUP = CHUNK // 16
CROWS = 126


def _body(c_hbm, a0_hbm, a1_hbm, a2_hbm, out_hbm,
          c_v, av0, av1, av2, rows_v,
          s_idx0, s_idx1, s_o0, s_o1):
    wid = lax.axis_index("s") * NC + lax.axis_index("c")
    s_idx = (s_idx0, s_idx1)
    s_o = (s_o0, s_o1)
    iota = lax.iota(jnp.int32, 16)

    pltpu.sync_copy(c_hbm, c_v)

    def stage_idx(c, p):
        off = pl.multiple_of(c * CHUNK, 8)
        pltpu.async_copy(a0_hbm.at[pl.ds(off, CHUNK)], av0.at[p], s_idx[p])
        pltpu.async_copy(a1_hbm.at[pl.ds(off, CHUNK)], av1.at[p], s_idx[p])
        pltpu.async_copy(a2_hbm.at[pl.ds(off, CHUNK)], av2.at[p], s_idx[p])

    def wait_idx(c, p):
        off = pl.multiple_of(c * CHUNK, 8)
        pltpu.make_async_copy(a0_hbm.at[pl.ds(off, CHUNK)], av0.at[p],
                              s_idx[p]).wait()
        pltpu.make_async_copy(a1_hbm.at[pl.ds(off, CHUNK)], av1.at[p],
                              s_idx[p]).wait()
        pltpu.make_async_copy(a2_hbm.at[pl.ds(off, CHUNK)], av2.at[p],
                              s_idx[p]).wait()

    def out_slice(c):
        return out_hbm.at[pl.ds(pl.multiple_of(c * CHUNK * EMB, 8),
                                CHUNK * EMB)]

    def half(i, h):
        k = i * 2 + h
        c = wid + NW * k
        p = h

        @pl.when(c < NCHUNKS)
        def _():
            wait_idx(c, p)
            @pl.when(c + NW < NCHUNKS)
            def _():
                stage_idx(c + NW, 1 - p)
            @pl.when(k >= 2)
            def _():
                pltpu.make_async_copy(rows_v.at[p], out_slice(c - 2 * NW),
                                      s_o[p]).wait()

            rows_p = rows_v.at[p]

            @plsc.parallel_loop(0, NGROUP, step=1, unroll=2)
            def group(g):
                g16 = pl.multiple_of(g * 16, 16)
                sl = pl.ds(g16, 16)
                c0 = jnp.clip(av0.at[p][sl], 0, 5)
                c1 = jnp.clip(av1.at[p][sl], 0, 6)
                c2 = jnp.clip(av2.at[p][sl], 0, 2)
                key = c0 + c1 * 6 + c2 * 42
                gkey = key * EMB
                sbase = (g16 + iota) * EMB
                for b in range(0, EMB, 16):
                    vs = [plsc.load_gather(c_v, [gkey + (b + j)])
                          for j in range(16)]
                    for j in range(16):
                        plsc.store_scatter(rows_p, [sbase + (b + j)], vs[j])

            pltpu.async_copy(rows_p, out_slice(c), s_o[p])

    stage_idx(wid, 0)

    def pair(i, carry):
        half(i, 0)
        half(i, 1)
        return carry

    lax.fori_loop(0, KMAX // 2, pair, 0)

    for k in (KMAX - 2, KMAX - 1):
        c = wid + NW * k

        @pl.when(c < NCHUNKS)
        def _():
            pltpu.make_async_copy(rows_v.at[k % 2], out_slice(c),
                                  s_o[k % 2]).wait()


@functools.partial(jax.jit, donate_argnums=())
def kernel(edge_attr, W0, W1, W2):
    comb = (W2[:, None, None, :] + W1[None, :, None, :]
            + W0[None, None, :, :]).reshape(CROWS * EMB)
    ea = edge_attr.astype(jnp.int32)
    a0, a1, a2 = ea[:, 0], ea[:, 1], ea[:, 2]

    run = pl.kernel(
        _body,
        out_type=jax.ShapeDtypeStruct((N_EDGES * EMB,), jnp.float32),
        mesh=plsc.VectorSubcoreMesh(core_axis_name="c", subcore_axis_name="s"),
        scratch_types=[
            pltpu.VMEM((CROWS * EMB,), jnp.float32),
            pltpu.VMEM((2, CHUNK), jnp.int32),
            pltpu.VMEM((2, CHUNK), jnp.int32),
            pltpu.VMEM((2, CHUNK), jnp.int32),
            pltpu.VMEM((2, CHUNK * EMB), jnp.float32),
            pltpu.SemaphoreType.DMA,
            pltpu.SemaphoreType.DMA,
            pltpu.SemaphoreType.DMA,
            pltpu.SemaphoreType.DMA,
        ],
        compiler_params=pltpu.CompilerParams(
            needs_layout_passes=False, use_tc_tiling_on_sc=False,
            disable_bounds_checks=True),
    )
    return run(comb, a0, a1, a2).reshape(N_EDGES, EMB)


# speedup vs baseline: 2.8261x; 2.8211x over previous
import functools

import jax
import jax.numpy as jnp
from jax import lax
from jax.experimental import pallas as pl
from jax.experimental.pallas import tpu as pltpu
from jax.experimental.pallas import tpu_sc as plsc

N_EDGES = 800000
EMB = 64
NC = 2
NS = 16
NW = NC * NS
CHUNK = 640
NCHUNKS = N_EDGES // CHUNK
KMAX = (NCHUNKS + NW - 1) // NW
NGROUP = CHUNK // 16
CROWS = 126


def _body(c_hbm, a0_hbm, a1_hbm, a2_hbm, out_hbm,
          c_v, av0, av1, av2, rows_v,
          s_idx0, s_idx1, s_o0, s_o1):
    wid = lax.axis_index("s") * NC + lax.axis_index("c")
    s_idx = (s_idx0, s_idx1)
    s_o = (s_o0, s_o1)
    iota = lax.iota(jnp.int32, 16)

    pltpu.sync_copy(c_hbm, c_v)

    def stage_idx(c, p):
        off = pl.multiple_of(c * CHUNK, 8)
        pltpu.async_copy(a0_hbm.at[pl.ds(off, CHUNK)], av0.at[p], s_idx[p])
        pltpu.async_copy(a1_hbm.at[pl.ds(off, CHUNK)], av1.at[p], s_idx[p])
        pltpu.async_copy(a2_hbm.at[pl.ds(off, CHUNK)], av2.at[p], s_idx[p])

    def wait_idx(c, p):
        off = pl.multiple_of(c * CHUNK, 8)
        pltpu.make_async_copy(a0_hbm.at[pl.ds(off, CHUNK)], av0.at[p],
                              s_idx[p]).wait()
        pltpu.make_async_copy(a1_hbm.at[pl.ds(off, CHUNK)], av1.at[p],
                              s_idx[p]).wait()
        pltpu.make_async_copy(a2_hbm.at[pl.ds(off, CHUNK)], av2.at[p],
                              s_idx[p]).wait()

    def out_slice(c):
        return out_hbm.at[pl.ds(pl.multiple_of(c * CHUNK * EMB, 8),
                                CHUNK * EMB)]

    def half(i, h):
        k = i * 2 + h
        c = wid + NW * k
        p = h

        @pl.when(c < NCHUNKS)
        def _():
            wait_idx(c, p)
            @pl.when(c + NW < NCHUNKS)
            def _():
                stage_idx(c + NW, 1 - p)
            @pl.when(k >= 2)
            def _():
                pltpu.make_async_copy(rows_v.at[p], out_slice(c - 2 * NW),
                                      s_o[p]).wait()

            rows_p = rows_v.at[p]

            def group(g, carry):
                g16 = pl.multiple_of(g * 16, 16)
                sl = pl.ds(g16, 16)
                c0 = jnp.clip(av0.at[p][sl], 0, 5)
                c1 = jnp.clip(av1.at[p][sl], 0, 6)
                c2 = jnp.clip(av2.at[p][sl], 0, 2)
                key = c0 + c1 * 6 + c2 * 42
                gkey = key * EMB
                sbase = (g16 + iota) * EMB
                for b in range(0, EMB, 16):
                    offs = [(iota + (b + j)) & (EMB - 1) for j in range(16)]
                    vs = [plsc.load_gather(c_v, [gkey + offs[j]])
                          for j in range(16)]
                    for j in range(16):
                        plsc.store_scatter(rows_p, [sbase + offs[j]], vs[j])
                return carry

            lax.fori_loop(0, NGROUP, group, 0)
            pltpu.async_copy(rows_p, out_slice(c), s_o[p])

    stage_idx(wid, 0)

    def pair(i, carry):
        half(i, 0)
        half(i, 1)
        return carry

    lax.fori_loop(0, KMAX // 2, pair, 0)

    for k in (KMAX - 2, KMAX - 1):
        c = wid + NW * k

        @pl.when(c < NCHUNKS)
        def _():
            pltpu.make_async_copy(rows_v.at[k % 2], out_slice(c),
                                  s_o[k % 2]).wait()


@functools.partial(jax.jit, donate_argnums=())
def kernel(edge_attr, W0, W1, W2):
    comb = (W2[:, None, None, :] + W1[None, :, None, :]
            + W0[None, None, :, :]).reshape(CROWS * EMB)
    ea = edge_attr.astype(jnp.int32)
    a0, a1, a2 = ea[:, 0], ea[:, 1], ea[:, 2]

    run = pl.kernel(
        _body,
        out_type=jax.ShapeDtypeStruct((N_EDGES * EMB,), jnp.float32),
        mesh=plsc.VectorSubcoreMesh(core_axis_name="c", subcore_axis_name="s"),
        scratch_types=[
            pltpu.VMEM((CROWS * EMB,), jnp.float32),
            pltpu.VMEM((2, CHUNK), jnp.int32),
            pltpu.VMEM((2, CHUNK), jnp.int32),
            pltpu.VMEM((2, CHUNK), jnp.int32),
            pltpu.VMEM((2, CHUNK * EMB), jnp.float32),
            pltpu.SemaphoreType.DMA,
            pltpu.SemaphoreType.DMA,
            pltpu.SemaphoreType.DMA,
            pltpu.SemaphoreType.DMA,
        ],
        compiler_params=pltpu.CompilerParams(
            needs_layout_passes=False, use_tc_tiling_on_sc=False,
            disable_bounds_checks=True),
    )
    return run(comb, a0, a1, a2).reshape(N_EDGES, EMB)


# speedup vs baseline: 2.8377x; 1.0041x over previous
import functools

import jax
import jax.numpy as jnp
from jax import lax
from jax.experimental import pallas as pl
from jax.experimental.pallas import tpu as pltpu
from jax.experimental.pallas import tpu_sc as plsc

N_EDGES = 800000
EMB = 64
NC = 2
NS = 16
NW = NC * NS
CHUNK = 640
NCHUNKS = N_EDGES // CHUNK
KMAX = (NCHUNKS + NW - 1) // NW
NGROUP = CHUNK // 16
CROWS = 126


def _body(c_hbm, a0_hbm, a1_hbm, a2_hbm, out_hbm,
          c_v, av0, av1, av2, rows_v,
          s_idx0, s_idx1, s_o0, s_o1):
    wid = lax.axis_index("s") * NC + lax.axis_index("c")
    s_idx = (s_idx0, s_idx1)
    s_o = (s_o0, s_o1)
    iota = lax.iota(jnp.int32, 16)

    pltpu.sync_copy(c_hbm, c_v)

    def stage_idx(c, p):
        off = pl.multiple_of(c * CHUNK, 8)
        pltpu.async_copy(a0_hbm.at[pl.ds(off, CHUNK)], av0.at[p], s_idx[p])
        pltpu.async_copy(a1_hbm.at[pl.ds(off, CHUNK)], av1.at[p], s_idx[p])
        pltpu.async_copy(a2_hbm.at[pl.ds(off, CHUNK)], av2.at[p], s_idx[p])

    def wait_idx(c, p):
        off = pl.multiple_of(c * CHUNK, 8)
        pltpu.make_async_copy(a0_hbm.at[pl.ds(off, CHUNK)], av0.at[p],
                              s_idx[p]).wait()
        pltpu.make_async_copy(a1_hbm.at[pl.ds(off, CHUNK)], av1.at[p],
                              s_idx[p]).wait()
        pltpu.make_async_copy(a2_hbm.at[pl.ds(off, CHUNK)], av2.at[p],
                              s_idx[p]).wait()

    def out_slice(c):
        return out_hbm.at[pl.ds(pl.multiple_of(c * CHUNK, 8), CHUNK)]

    def half(i, h):
        k = i * 2 + h
        c = wid + NW * k
        p = h

        @pl.when(c < NCHUNKS)
        def _():
            wait_idx(c, p)
            @pl.when(c + NW < NCHUNKS)
            def _():
                stage_idx(c + NW, 1 - p)
            @pl.when(k >= 2)
            def _():
                pltpu.make_async_copy(rows_v.at[p], out_slice(c - 2 * NW),
                                      s_o[p]).wait()

            rows_p = rows_v.at[p]

            def group(g, carry):
                g16 = pl.multiple_of(g * 16, 16)
                sl = pl.ds(g16, 16)
                c0 = jnp.clip(av0.at[p][sl], 0, 5)
                c1 = jnp.clip(av1.at[p][sl], 0, 6)
                c2 = jnp.clip(av2.at[p][sl], 0, 2)
                key = c0 + c1 * 6 + c2 * 42
                gkey = key * EMB
                srow = g16 + iota
                for b in range(0, EMB, 16):
                    offs = [(iota + (b + j)) & (EMB - 1) for j in range(16)]
                    vs = [plsc.load_gather(c_v, [gkey + offs[j]])
                          for j in range(16)]
                    for j in range(16):
                        plsc.store_scatter(rows_p, [srow, offs[j]], vs[j])
                return carry

            lax.fori_loop(0, NGROUP, group, 0)
            pltpu.async_copy(rows_p, out_slice(c), s_o[p])

    stage_idx(wid, 0)

    def pair(i, carry):
        half(i, 0)
        half(i, 1)
        return carry

    lax.fori_loop(0, KMAX // 2, pair, 0)

    for k in (KMAX - 2, KMAX - 1):
        c = wid + NW * k

        @pl.when(c < NCHUNKS)
        def _():
            pltpu.make_async_copy(rows_v.at[k % 2], out_slice(c),
                                  s_o[k % 2]).wait()


@functools.partial(jax.jit, donate_argnums=())
def kernel(edge_attr, W0, W1, W2):
    comb = (W2[:, None, None, :] + W1[None, :, None, :]
            + W0[None, None, :, :]).reshape(CROWS * EMB)
    ea = edge_attr.astype(jnp.int32)
    a0, a1, a2 = ea[:, 0], ea[:, 1], ea[:, 2]

    run = pl.kernel(
        _body,
        out_type=jax.ShapeDtypeStruct((N_EDGES, EMB), jnp.float32),
        mesh=plsc.VectorSubcoreMesh(core_axis_name="c", subcore_axis_name="s"),
        scratch_types=[
            pltpu.VMEM((CROWS * EMB,), jnp.float32),
            pltpu.VMEM((2, CHUNK), jnp.int32),
            pltpu.VMEM((2, CHUNK), jnp.int32),
            pltpu.VMEM((2, CHUNK), jnp.int32),
            pltpu.VMEM((2, CHUNK, EMB), jnp.float32),
            pltpu.SemaphoreType.DMA,
            pltpu.SemaphoreType.DMA,
            pltpu.SemaphoreType.DMA,
            pltpu.SemaphoreType.DMA,
        ],
        compiler_params=pltpu.CompilerParams(
            needs_layout_passes=False, use_tc_tiling_on_sc=False,
            disable_bounds_checks=True),
    )
    return run(comb, a0, a1, a2)


# speedup vs baseline: 4.0795x; 1.4376x over previous
import functools

import jax
import jax.numpy as jnp
from jax import lax
from jax.experimental import pallas as pl
from jax.experimental.pallas import tpu as pltpu
from jax.experimental.pallas import tpu_sc as plsc

N_EDGES = 800000
EMB = 64
NC = 2
NS = 16
NW = NC * NS
CHUNK = 320
NCHUNKS = N_EDGES // CHUNK
KMAX = (NCHUNKS + NW - 1) // NW
NGROUP = CHUNK // 16
CROWS = 126


def _body(c_hbm, a0_hbm, a1_hbm, a2_hbm, out_hbm,
          c_v, av0a, av0b, av1a, av1b, av2a, av2b, rows_v,
          s_idx0, s_idx1, s_o0, s_o1):
    wid = lax.axis_index("s") * NC + lax.axis_index("c")
    av0 = (av0a, av0b)
    av1 = (av1a, av1b)
    av2 = (av2a, av2b)
    s_idx = (s_idx0, s_idx1)
    s_o = (s_o0, s_o1)
    iota = lax.iota(jnp.int32, 16)

    pltpu.sync_copy(c_hbm, c_v)

    def stage_idx(c, p):
        off = pl.multiple_of(c * CHUNK, 8)
        pltpu.async_copy(a0_hbm.at[pl.ds(off, CHUNK)], av0[p], s_idx[p])
        pltpu.async_copy(a1_hbm.at[pl.ds(off, CHUNK)], av1[p], s_idx[p])
        pltpu.async_copy(a2_hbm.at[pl.ds(off, CHUNK)], av2[p], s_idx[p])

    def wait_idx(c, p):
        off = pl.multiple_of(c * CHUNK, 8)
        pltpu.make_async_copy(a0_hbm.at[pl.ds(off, CHUNK)], av0[p],
                              s_idx[p]).wait()
        pltpu.make_async_copy(a1_hbm.at[pl.ds(off, CHUNK)], av1[p],
                              s_idx[p]).wait()
        pltpu.make_async_copy(a2_hbm.at[pl.ds(off, CHUNK)], av2[p],
                              s_idx[p]).wait()

    def out_slice(c):
        return out_hbm.at[pl.ds(pl.multiple_of(c * CHUNK, 8), CHUNK)]

    def half(i, h):
        k = i * 2 + h
        c = wid + NW * k
        p = h

        @pl.when(c < NCHUNKS)
        def _():
            wait_idx(c, p)
            @pl.when(c + NW < NCHUNKS)
            def _():
                stage_idx(c + NW, 1 - p)
            @pl.when(k >= 2)
            def _():
                pltpu.make_async_copy(rows_v.at[p], out_slice(c - 2 * NW),
                                      s_o[p]).wait()

            rows_p = rows_v.at[p]

            def group(g, carry):
                g16 = pl.multiple_of(g * 16, 16)
                sl = pl.ds(g16, 16)
                c0 = jnp.clip(av0[p][sl], 0, 5)
                c1 = jnp.clip(av1[p][sl], 0, 6)
                c2 = jnp.clip(av2[p][sl], 0, 2)
                key = c0 + c1 * 6 + c2 * 42
                gkey = key * EMB
                srow = g16 + iota
                for b in range(0, EMB, 16):
                    offs = [(iota + (b + j)) & (EMB - 1) for j in range(16)]
                    vs = [plsc.load_gather(c_v, [gkey + offs[j]])
                          for j in range(16)]
                    for j in range(16):
                        plsc.store_scatter(rows_p, [srow, offs[j]], vs[j])
                return carry

            lax.fori_loop(0, NGROUP, group, 0)
            pltpu.async_copy(rows_p, out_slice(c), s_o[p])

    stage_idx(wid, 0)

    def pair(i, carry):
        half(i, 0)
        half(i, 1)
        return carry

    lax.fori_loop(0, (KMAX + 1) // 2, pair, 0)

    kk = (KMAX + 1) // 2 * 2
    for k in (kk - 2, kk - 1):
        c = wid + NW * k

        @pl.when(c < NCHUNKS)
        def _():
            pltpu.make_async_copy(rows_v.at[k % 2], out_slice(c),
                                  s_o[k % 2]).wait()


@functools.partial(jax.jit, donate_argnums=())
def kernel(edge_attr, W0, W1, W2):
    comb = (W2[:, None, None, :] + W1[None, :, None, :]
            + W0[None, None, :, :]).reshape(CROWS * EMB)
    ea = edge_attr.astype(jnp.int32)
    a0, a1, a2 = ea[:, 0], ea[:, 1], ea[:, 2]

    run = pl.kernel(
        _body,
        out_type=jax.ShapeDtypeStruct((N_EDGES, EMB), jnp.float32),
        mesh=plsc.VectorSubcoreMesh(core_axis_name="c", subcore_axis_name="s"),
        scratch_types=[
            pltpu.VMEM((CROWS * EMB,), jnp.float32),
            pltpu.VMEM((CHUNK,), jnp.int32),
            pltpu.VMEM((CHUNK,), jnp.int32),
            pltpu.VMEM((CHUNK,), jnp.int32),
            pltpu.VMEM((CHUNK,), jnp.int32),
            pltpu.VMEM((CHUNK,), jnp.int32),
            pltpu.VMEM((CHUNK,), jnp.int32),
            pltpu.VMEM((2, CHUNK, EMB), jnp.float32),
            pltpu.SemaphoreType.DMA,
            pltpu.SemaphoreType.DMA,
            pltpu.SemaphoreType.DMA,
            pltpu.SemaphoreType.DMA,
        ],
        compiler_params=pltpu.CompilerParams(
            needs_layout_passes=False, use_tc_tiling_on_sc=True,
            disable_bounds_checks=True),
    )
    return run(comb, a0, a1, a2)
